# Initial kernel scaffold; baseline (speedup 1.0000x reference)
import jax
import jax.numpy as jnp
from jax.experimental import pallas as pl


def kernel(x, edge_index, W1_l, b1_l, W1_r, W2_l, b2_l, W2_r):
    raise NotImplementedError("write your pallas kernel here")


# speedup vs baseline: 6.7431x; 6.7431x over previous
import functools

import jax
import jax.numpy as jnp
from jax import lax
from jax.experimental import pallas as pl
from jax.experimental.pallas import tpu as pltpu
from jax.experimental.pallas import tpu_sc as plsc

_N = 10000
_D = 128
_E = 320000
_NC = 2
_NS = 16
_NW = _NC * _NS
_EPW = _E // _NW
_CH = 80
_NCH = _EPW // _CH
_NB = 5
_CB = _NCH // _NB
_NP = 10240
_RPS = _NP // _NS

_mesh = plsc.VectorSubcoreMesh(core_axis_name="c", subcore_axis_name="s")


def _fill_rows(rows, value):
  def fill(j, carry):
    for k in range(_D // 16):
      rows[j, pl.ds(k * 16, 16)] = jnp.full((16,), value, jnp.float32)
    return carry

  lax.fori_loop(0, _CH, fill, 0)


@functools.partial(
    pl.kernel,
    out_type=[jax.ShapeDtypeStruct((_NC, _NP, _D), jnp.float32)],
    mesh=_mesh,
    scratch_types=[
        pltpu.VMEM((_CB, _CH), jnp.int32),
        pltpu.VMEM((_CB, _CH), jnp.int32),
        pltpu.VMEM((_CH, _D), jnp.float32),
        pltpu.SemaphoreType.DMA,
        pltpu.VMEM_SHARED((_NP, _D), jnp.float32),
    ],
)
def _agg(src_h, dst_h, x_h, agg_o, idx_s, idx_d, rows, sem, agg_sh):
  c = lax.axis_index("c")
  s = lax.axis_index("s")
  r0 = s * _RPS

  _fill_rows(rows, 0.0)
  for k in range(_RPS // _CH):
    pltpu.sync_copy(rows, agg_sh.at[pl.ds(r0 + k * _CH, _CH)])
  plsc.subcore_barrier()

  def step(j, carry):
    pltpu.async_copy(x_h.at[idx_s.at[j]], rows, sem).wait()
    pltpu.sync_copy(rows, agg_sh.at[idx_d.at[j]], add=True)
    return carry

  for b in range(_NB):
    pltpu.sync_copy(src_h.at[c, s, b], idx_s)
    pltpu.sync_copy(dst_h.at[c, s, b], idx_d)
    lax.fori_loop(0, _CB, step, 0)

  plsc.subcore_barrier()
  pltpu.sync_copy(agg_sh.at[pl.ds(r0, _RPS)], agg_o.at[c, pl.ds(r0, _RPS)])


@functools.partial(
    pl.kernel,
    out_type=[jax.ShapeDtypeStruct((_NC, _NP, _D), jnp.float32)],
    mesh=_mesh,
    scratch_types=[
        pltpu.VMEM((_CB, _CH), jnp.int32),
        pltpu.VMEM((_CH, _D), jnp.float32),
        pltpu.VMEM_SHARED((_NP, _D), jnp.float32),
    ],
)
def _cnt(dst_h, cnt_o, idx_d, rows, cnt_sh):
  c = lax.axis_index("c")
  s = lax.axis_index("s")
  r0 = s * _RPS

  _fill_rows(rows, 0.0)
  for k in range(_RPS // _CH):
    pltpu.sync_copy(rows, cnt_sh.at[pl.ds(r0 + k * _CH, _CH)])
  plsc.subcore_barrier()
  _fill_rows(rows, 1.0)

  def step(j, carry):
    pltpu.sync_copy(rows, cnt_sh.at[idx_d.at[j]], add=True)
    return carry

  for b in range(_NB):
    pltpu.sync_copy(dst_h.at[c, s, b], idx_d)
    lax.fori_loop(0, _CB, step, 0)

  plsc.subcore_barrier()
  pltpu.sync_copy(cnt_sh.at[pl.ds(r0, _RPS)], cnt_o.at[c, pl.ds(r0, _RPS)])


_BLK = 1000


def _dense_body(p0, p1, c0, c1, x, wl, wr, b, o, *, relu):
  cnt = jnp.maximum(c0[...] + c1[...], 1.0)
  agg = (p0[...] + p1[...]) / cnt
  acc = lax.dot_general(agg, wl[...], (((1,), (1,)), ((), ())),
                        preferred_element_type=jnp.float32)
  acc = acc + lax.dot_general(x[...], wr[...], (((1,), (1,)), ((), ())),
                              preferred_element_type=jnp.float32)
  acc = acc + b[...]
  o[...] = jnp.maximum(acc, 0.0) if relu else acc


def _make_dense(relu):
  row_spec = pl.BlockSpec((_BLK, _D), lambda i: (i, 0))
  return pl.pallas_call(
      functools.partial(_dense_body, relu=relu),
      grid=(_N // _BLK,),
      in_specs=[
          row_spec,
          row_spec,
          row_spec,
          row_spec,
          row_spec,
          pl.BlockSpec((_D, _D), lambda i: (0, 0)),
          pl.BlockSpec((_D, _D), lambda i: (0, 0)),
          pl.BlockSpec((1, _D), lambda i: (0, 0)),
      ],
      out_specs=row_spec,
      out_shape=jax.ShapeDtypeStruct((_N, _D), jnp.float32),
  )


_dense_relu = _make_dense(True)
_dense_plain = _make_dense(False)


def kernel(x, edge_index, W1_l, b1_l, W1_r, W2_l, b2_l, W2_r):
  src = edge_index[0].astype(jnp.int32).reshape(_NC, _NS, _NB, _CB, _CH)
  dst = edge_index[1].astype(jnp.int32).reshape(_NC, _NS, _NB, _CB, _CH)

  (cnt,) = _cnt(dst)
  (agg1,) = _agg(src, dst, x)
  h = _dense_relu(agg1[0], agg1[1], cnt[0], cnt[1], x,
                  W1_l, W1_r, b1_l.reshape(1, _D))
  (agg2,) = _agg(src, dst, h)
  out = _dense_plain(agg2[0], agg2[1], cnt[0], cnt[1], h,
                     W2_l, W2_r, b2_l.reshape(1, _D))
  return out
